# baseline (speedup 1.0000x reference)
import jax
import jax.numpy as jnp
import numpy as np
from jax.experimental import pallas as pl

OMEGA = 30.0
L_FREQ = 6
NUM_EXPERTS = 8
HID = 128
TILE = 1024


def _dot(a, b):
    return jnp.dot(a, b, preferred_element_type=jnp.float32)


def _dotx(a, b):
    return jnp.dot(a, b, preferred_element_type=jnp.float32,
                   precision=jax.lax.Precision.HIGHEST)


def _topk2_weights(p):
    iota8 = jax.lax.broadcasted_iota(jnp.int32, p.shape, 1)
    m1 = jnp.max(p, axis=1, keepdims=True)
    i1 = jnp.min(jnp.where(p == m1, iota8, NUM_EXPERTS), axis=1, keepdims=True)
    mask1 = iota8 == i1
    p2 = jnp.where(mask1, -1.0, p)
    m2 = jnp.max(p2, axis=1, keepdims=True)
    i2 = jnp.min(jnp.where(p2 == m2, iota8, NUM_EXPERTS), axis=1, keepdims=True)
    mask = mask1 | (iota8 == i2)
    masked = jnp.where(mask, p, 0.0)
    return masked / (jnp.sum(masked, axis=1, keepdims=True) + 1e-9)


def _body(coords_ref, pmat_ref, cmask_ref, emat_ref,
          eW0, eb0, eW1, eb1, eW2, eb2, eW3, eb3,
          gW0, gb0, gW1, gb1, gW2, gb2, gW3, gb3,
          vemb, vproj,
          rW0, rb0, rW1, rb1, rW2, rb2, rW3, rb3,
          sW0, sb0, sW1, sb1, sW2, sb2, sW3, sb3,
          rgb_out, sig_out):
    c = coords_ref[:]

    f12 = pmat_ref[:]
    ph = jnp.concatenate([c[:, d:d + 1] * f12 for d in range(3)], axis=1)
    trig = jnp.where(cmask_ref[:] > 0.0, jnp.cos(ph), jnp.sin(ph))

    wexp = []
    for v in range(2):
        ve = vemb[v:v + 1, :]
        g = jnp.concatenate(
            [c, jnp.broadcast_to(ve, (c.shape[0], ve.shape[1]))], axis=1)
        g = jnp.sin(OMEGA * (_dot(g, gW0[:]) + gb0[:]))
        g = jnp.sin(OMEGA * (_dot(g, gW1[:]) + gb1[:]))
        g = jnp.sin(OMEGA * (_dot(g, gW2[:]) + gb2[:]))
        logits = _dot(g, gW3[:]) + gb3[:]
        m = jnp.max(logits, axis=1, keepdims=True)
        ex = jnp.exp(logits - m)
        p = ex / jnp.sum(ex, axis=1, keepdims=True)
        w = _topk2_weights(p)
        wexp.append(_dotx(w, emat_ref[:]))

    pe = jnp.concatenate([c, trig], axis=1)
    acc0 = jnp.zeros((c.shape[0], HID), jnp.float32)
    acc1 = jnp.zeros((c.shape[0], HID), jnp.float32)
    for e in range(NUM_EXPERTS):
        lo = e * HID
        h = jnp.sin(OMEGA * (_dot(pe, eW0[e]) + eb0[e:e + 1, :]))
        h = jnp.sin(OMEGA * (_dot(h, eW1[e]) + eb1[e:e + 1, :]))
        h = jnp.sin(OMEGA * (_dot(h, eW2[e]) + eb2[e:e + 1, :]))
        f = _dot(h, eW3[e]) + eb3[e:e + 1, :]
        acc0 = acc0 + f * wexp[0][:, lo:lo + HID]
        acc1 = acc1 + f * wexp[1][:, lo:lo + HID]

    vp = _dot(vemb[:], vproj[:])

    for v, (acc, dW0, db0, dW1, db1, dW2, db2, dW3, db3, out_ref) in enumerate((
            (acc0, rW0, rb0, rW1, rb1, rW2, rb2, rW3, rb3, rgb_out),
            (acc1, sW0, sb0, sW1, sb1, sW2, sb2, sW3, sb3, sig_out))):
        hv = acc + vp[v:v + 1, :]
        d0 = jnp.sin(OMEGA * (_dot(hv, dW0[:]) + db0[:]))
        d0 = jnp.sin(OMEGA * (_dot(d0, dW1[:]) + db1[:]))
        d0 = jnp.sin(OMEGA * (_dot(d0, dW2[:]) + db2[:]))
        out_ref[:] = _dot(d0, dW3[:]) + db3[:]


def kernel(coords, expert_W0, expert_b0, expert_W1, expert_b1, expert_W2, expert_b2,
           expert_W3, expert_b3, gate_W0, gate_b0, gate_W1, gate_b1, gate_W2, gate_b2,
           gate_W3, gate_b3, view_embedding, view_embed_proj,
           dec_rgb_W0, dec_rgb_b0, dec_rgb_W1, dec_rgb_b1, dec_rgb_W2, dec_rgb_b2,
           dec_rgb_W3, dec_rgb_b3, dec_sigma_W0, dec_sigma_b0, dec_sigma_W1, dec_sigma_b1,
           dec_sigma_W2, dec_sigma_b2, dec_sigma_W3, dec_sigma_b3):
    n = coords.shape[0]

    freqs = (2.0 ** np.arange(L_FREQ, dtype=np.float32)) * np.pi
    pmat = np.concatenate([freqs, freqs])[None, :]
    cmask = np.zeros((1, 36), np.float32)
    for d in range(3):
        cmask[0, 12 * d + 6:12 * d + 12] = 1.0
    emat = np.zeros((NUM_EXPERTS, NUM_EXPERTS * HID), np.float32)
    for e in range(NUM_EXPERTS):
        emat[e, e * HID:(e + 1) * HID] = 1.0

    def r2(x):
        return x.reshape(1, -1)

    ins = (coords, jnp.asarray(pmat), jnp.asarray(cmask), jnp.asarray(emat),
           expert_W0, expert_b0, expert_W1, expert_b1, expert_W2, expert_b2,
           expert_W3, expert_b3,
           gate_W0, r2(gate_b0), gate_W1, r2(gate_b1), gate_W2, r2(gate_b2),
           gate_W3, r2(gate_b3), view_embedding, view_embed_proj,
           dec_rgb_W0, r2(dec_rgb_b0), dec_rgb_W1, r2(dec_rgb_b1),
           dec_rgb_W2, r2(dec_rgb_b2), dec_rgb_W3, r2(dec_rgb_b3),
           dec_sigma_W0, r2(dec_sigma_b0), dec_sigma_W1, r2(dec_sigma_b1),
           dec_sigma_W2, r2(dec_sigma_b2), dec_sigma_W3, r2(dec_sigma_b3))

    grid = (n // TILE,)

    def const_spec(x):
        nd = x.ndim
        return pl.BlockSpec(x.shape, lambda i, _nd=nd: (0,) * _nd)

    in_specs = [pl.BlockSpec((TILE, 3), lambda i: (i, 0))] + [const_spec(x) for x in ins[1:]]
    out_specs = [pl.BlockSpec((TILE, 3), lambda i: (i, 0)),
                 pl.BlockSpec((TILE, 1), lambda i: (i, 0))]
    out_shape = [jax.ShapeDtypeStruct((n, 3), jnp.float32),
                 jax.ShapeDtypeStruct((n, 1), jnp.float32)]

    rgb, sigma = pl.pallas_call(
        _body,
        grid=grid,
        in_specs=in_specs,
        out_specs=out_specs,
        out_shape=out_shape,
    )(*ins)
    return (rgb, sigma)


# speedup vs baseline: 2.7843x; 2.7843x over previous
import jax
import jax.numpy as jnp
import numpy as np
from jax.experimental import pallas as pl

OMEGA = 30.0
L_FREQ = 6
NUM_EXPERTS = 8
HID = 128
TILE = 1024


def _dot(a, b):
    return jnp.dot(a, b, preferred_element_type=jnp.float32)


def _dotx(a, b):
    return jnp.dot(a, b, preferred_element_type=jnp.float32,
                   precision=jax.lax.Precision.HIGHEST)


_FSIN_C = (6.283182621002197, -41.34142303466797, 81.59618377685547,
           -76.58012390136719, 41.205482482910156, -12.271398544311523)
_INV2PI = 0.15915494309644432
_MAGIC = 12582912.0


def _fsin(x):
    r0 = x * _INV2PI
    r = r0 - jnp.round(r0)
    u = r * r
    p = _FSIN_C[5]
    p = p * u + _FSIN_C[4]
    p = p * u + _FSIN_C[3]
    p = p * u + _FSIN_C[2]
    p = p * u + _FSIN_C[1]
    p = p * u + _FSIN_C[0]
    return p * r


def _topk2_weights(p):
    iota8 = jax.lax.broadcasted_iota(jnp.int32, p.shape, 1)
    m1 = jnp.max(p, axis=1, keepdims=True)
    i1 = jnp.min(jnp.where(p == m1, iota8, NUM_EXPERTS), axis=1, keepdims=True)
    mask1 = iota8 == i1
    p2 = jnp.where(mask1, -1.0, p)
    m2 = jnp.max(p2, axis=1, keepdims=True)
    i2 = jnp.min(jnp.where(p2 == m2, iota8, NUM_EXPERTS), axis=1, keepdims=True)
    mask = mask1 | (iota8 == i2)
    masked = jnp.where(mask, p, 0.0)
    return masked / (jnp.sum(masked, axis=1, keepdims=True) + 1e-9)


def _body(coords_ref, pmat_ref, cmask_ref, emat_ref,
          eW0, eb0, eW1, eb1, eW2, eb2, eW3, eb3,
          gW0, gb0, gW1, gb1, gW2, gb2, gW3, gb3,
          vemb, vproj,
          rW0, rb0, rW1, rb1, rW2, rb2, rW3, rb3,
          sW0, sb0, sW1, sb1, sW2, sb2, sW3, sb3,
          rgb_out, sig_out):
    c = coords_ref[:]

    f12 = pmat_ref[:]
    ph = jnp.concatenate([c[:, d:d + 1] * f12 for d in range(3)], axis=1)
    trig = jnp.where(cmask_ref[:] > 0.0, jnp.cos(ph), jnp.sin(ph))

    wexp = []
    for v in range(2):
        ve = vemb[v:v + 1, :]
        g = jnp.concatenate(
            [c, jnp.broadcast_to(ve, (c.shape[0], ve.shape[1]))], axis=1)
        g = jnp.sin(OMEGA * (_dot(g, gW0[:]) + gb0[:]))
        g = jnp.sin(OMEGA * (_dot(g, gW1[:]) + gb1[:]))
        g = jnp.sin(OMEGA * (_dot(g, gW2[:]) + gb2[:]))
        logits = _dot(g, gW3[:]) + gb3[:]
        m = jnp.max(logits, axis=1, keepdims=True)
        ex = jnp.exp(logits - m)
        p = ex / jnp.sum(ex, axis=1, keepdims=True)
        w = _topk2_weights(p)
        wexp.append(_dotx(w, emat_ref[:]))

    pe = jnp.concatenate([c, trig], axis=1)
    acc0 = jnp.zeros((c.shape[0], HID), jnp.float32)
    acc1 = jnp.zeros((c.shape[0], HID), jnp.float32)
    for e in range(NUM_EXPERTS):
        lo = e * HID
        h = _fsin(OMEGA * (_dot(pe, eW0[e]) + eb0[e:e + 1, :]))
        h = _fsin(OMEGA * (_dot(h, eW1[e]) + eb1[e:e + 1, :]))
        h = _fsin(OMEGA * (_dot(h, eW2[e]) + eb2[e:e + 1, :]))
        f = _dot(h, eW3[e]) + eb3[e:e + 1, :]
        acc0 = acc0 + f * wexp[0][:, lo:lo + HID]
        acc1 = acc1 + f * wexp[1][:, lo:lo + HID]

    vp = _dot(vemb[:], vproj[:])

    for v, (acc, dW0, db0, dW1, db1, dW2, db2, dW3, db3, out_ref) in enumerate((
            (acc0, rW0, rb0, rW1, rb1, rW2, rb2, rW3, rb3, rgb_out),
            (acc1, sW0, sb0, sW1, sb1, sW2, sb2, sW3, sb3, sig_out))):
        hv = acc + vp[v:v + 1, :]
        d0 = _fsin(OMEGA * (_dot(hv, dW0[:]) + db0[:]))
        d0 = _fsin(OMEGA * (_dot(d0, dW1[:]) + db1[:]))
        d0 = _fsin(OMEGA * (_dot(d0, dW2[:]) + db2[:]))
        out_ref[:] = _dot(d0, dW3[:]) + db3[:]


def kernel(coords, expert_W0, expert_b0, expert_W1, expert_b1, expert_W2, expert_b2,
           expert_W3, expert_b3, gate_W0, gate_b0, gate_W1, gate_b1, gate_W2, gate_b2,
           gate_W3, gate_b3, view_embedding, view_embed_proj,
           dec_rgb_W0, dec_rgb_b0, dec_rgb_W1, dec_rgb_b1, dec_rgb_W2, dec_rgb_b2,
           dec_rgb_W3, dec_rgb_b3, dec_sigma_W0, dec_sigma_b0, dec_sigma_W1, dec_sigma_b1,
           dec_sigma_W2, dec_sigma_b2, dec_sigma_W3, dec_sigma_b3):
    n = coords.shape[0]

    freqs = (2.0 ** np.arange(L_FREQ, dtype=np.float32)) * np.pi
    pmat = np.concatenate([freqs, freqs])[None, :]
    cmask = np.zeros((1, 36), np.float32)
    for d in range(3):
        cmask[0, 12 * d + 6:12 * d + 12] = 1.0
    emat = np.zeros((NUM_EXPERTS, NUM_EXPERTS * HID), np.float32)
    for e in range(NUM_EXPERTS):
        emat[e, e * HID:(e + 1) * HID] = 1.0

    def r2(x):
        return x.reshape(1, -1)

    ins = (coords, jnp.asarray(pmat), jnp.asarray(cmask), jnp.asarray(emat),
           expert_W0, expert_b0, expert_W1, expert_b1, expert_W2, expert_b2,
           expert_W3, expert_b3,
           gate_W0, r2(gate_b0), gate_W1, r2(gate_b1), gate_W2, r2(gate_b2),
           gate_W3, r2(gate_b3), view_embedding, view_embed_proj,
           dec_rgb_W0, r2(dec_rgb_b0), dec_rgb_W1, r2(dec_rgb_b1),
           dec_rgb_W2, r2(dec_rgb_b2), dec_rgb_W3, r2(dec_rgb_b3),
           dec_sigma_W0, r2(dec_sigma_b0), dec_sigma_W1, r2(dec_sigma_b1),
           dec_sigma_W2, r2(dec_sigma_b2), dec_sigma_W3, r2(dec_sigma_b3))

    grid = (n // TILE,)

    def const_spec(x):
        nd = x.ndim
        return pl.BlockSpec(x.shape, lambda i, _nd=nd: (0,) * _nd)

    in_specs = [pl.BlockSpec((TILE, 3), lambda i: (i, 0))] + [const_spec(x) for x in ins[1:]]
    out_specs = [pl.BlockSpec((TILE, 3), lambda i: (i, 0)),
                 pl.BlockSpec((TILE, 1), lambda i: (i, 0))]
    out_shape = [jax.ShapeDtypeStruct((n, 3), jnp.float32),
                 jax.ShapeDtypeStruct((n, 1), jnp.float32)]

    rgb, sigma = pl.pallas_call(
        _body,
        grid=grid,
        in_specs=in_specs,
        out_specs=out_specs,
        out_shape=out_shape,
    )(*ins)
    return (rgb, sigma)


# speedup vs baseline: 3.3711x; 1.2108x over previous
import jax
import jax.numpy as jnp
import numpy as np
from jax.experimental import pallas as pl

OMEGA = 30.0
L_FREQ = 6
NUM_EXPERTS = 8
HID = 128
TILE = 1024


def _dot(a, b):
    return jnp.dot(a, b, preferred_element_type=jnp.float32)


def _dotx(a, b):
    return jnp.dot(a, b, preferred_element_type=jnp.float32,
                   precision=jax.lax.Precision.HIGHEST)


_FSIN_C = (6.283182621002197, -41.34142303466797, 81.59618377685547,
           -76.58012390136719, 41.205482482910156, -12.271398544311523)
_INV2PI = 0.15915494309644432
_MAGIC = 12582912.0


def _fsin(x):
    r0 = x * _INV2PI
    return _fsin_r(r0 - jnp.round(r0))


def _fsin_r(r):
    u = r * r
    p = _FSIN_C[5]
    p = p * u + _FSIN_C[4]
    p = p * u + _FSIN_C[3]
    p = p * u + _FSIN_C[2]
    p = p * u + _FSIN_C[1]
    p = p * u + _FSIN_C[0]
    return p * r


def _topk2_weights(p):
    iota8 = jax.lax.broadcasted_iota(jnp.int32, p.shape, 1)
    m1 = jnp.max(p, axis=1, keepdims=True)
    i1 = jnp.min(jnp.where(p == m1, iota8, NUM_EXPERTS), axis=1, keepdims=True)
    mask1 = iota8 == i1
    p2 = jnp.where(mask1, -1.0, p)
    m2 = jnp.max(p2, axis=1, keepdims=True)
    i2 = jnp.min(jnp.where(p2 == m2, iota8, NUM_EXPERTS), axis=1, keepdims=True)
    mask = mask1 | (iota8 == i2)
    masked = jnp.where(mask, p, 0.0)
    return masked / (jnp.sum(masked, axis=1, keepdims=True) + 1e-9)


def _body(coords_ref, pmat_ref, cmask_ref,
          eW0, eb0, eW1, eb1, eW2, eb2, eW3, eb3,
          gW0, gb0, gW1, gb1, gW2, gb2, gW3, gb3,
          vemb, vproj,
          rW0, rb0, rW1, rb1, rW2, rb2, rW3, rb3,
          sW0, sb0, sW1, sb1, sW2, sb2, sW3, sb3,
          rgb_out, sig_out):
    c = coords_ref[:]

    f12 = pmat_ref[:]
    ph = jnp.concatenate([c[:, d:d + 1] * f12 for d in range(3)], axis=1)
    r0 = ph * _INV2PI
    r0 = jnp.where(cmask_ref[:] > 0.0, r0 + 0.25, r0)
    trig = _fsin_r(r0 - jnp.round(r0))

    wexp = []
    for v in range(2):
        ve = vemb[v:v + 1, :]
        g = jnp.concatenate(
            [c, jnp.broadcast_to(ve, (c.shape[0], ve.shape[1]))], axis=1)
        g = jnp.sin(OMEGA * (_dot(g, gW0[:]) + gb0[:]))
        g = jnp.sin(OMEGA * (_dot(g, gW1[:]) + gb1[:]))
        g = jnp.sin(OMEGA * (_dot(g, gW2[:]) + gb2[:]))
        logits = _dot(g, gW3[:]) + gb3[:]
        m = jnp.max(logits, axis=1, keepdims=True)
        ex = jnp.exp(logits - m)
        p = ex / jnp.sum(ex, axis=1, keepdims=True)
        wexp.append(_topk2_weights(p))

    pe = jnp.concatenate([c, trig], axis=1)
    acc0 = jnp.zeros((c.shape[0], HID), jnp.float32)
    acc1 = jnp.zeros((c.shape[0], HID), jnp.float32)
    for e in range(NUM_EXPERTS):
        h = _fsin(OMEGA * (_dot(pe, eW0[e]) + eb0[e:e + 1, :]))
        h = _fsin(OMEGA * (_dot(h, eW1[e]) + eb1[e:e + 1, :]))
        h = _fsin(OMEGA * (_dot(h, eW2[e]) + eb2[e:e + 1, :]))
        f = _dot(h, eW3[e]) + eb3[e:e + 1, :]
        acc0 = acc0 + f * wexp[0][:, e:e + 1]
        acc1 = acc1 + f * wexp[1][:, e:e + 1]

    vp = _dot(vemb[:], vproj[:])

    for v, (acc, dW0, db0, dW1, db1, dW2, db2, dW3, db3, out_ref) in enumerate((
            (acc0, rW0, rb0, rW1, rb1, rW2, rb2, rW3, rb3, rgb_out),
            (acc1, sW0, sb0, sW1, sb1, sW2, sb2, sW3, sb3, sig_out))):
        hv = acc + vp[v:v + 1, :]
        d0 = _fsin(OMEGA * (_dot(hv, dW0[:]) + db0[:]))
        d0 = _fsin(OMEGA * (_dot(d0, dW1[:]) + db1[:]))
        d0 = _fsin(OMEGA * (_dot(d0, dW2[:]) + db2[:]))
        out_ref[:] = _dot(d0, dW3[:]) + db3[:]


def kernel(coords, expert_W0, expert_b0, expert_W1, expert_b1, expert_W2, expert_b2,
           expert_W3, expert_b3, gate_W0, gate_b0, gate_W1, gate_b1, gate_W2, gate_b2,
           gate_W3, gate_b3, view_embedding, view_embed_proj,
           dec_rgb_W0, dec_rgb_b0, dec_rgb_W1, dec_rgb_b1, dec_rgb_W2, dec_rgb_b2,
           dec_rgb_W3, dec_rgb_b3, dec_sigma_W0, dec_sigma_b0, dec_sigma_W1, dec_sigma_b1,
           dec_sigma_W2, dec_sigma_b2, dec_sigma_W3, dec_sigma_b3):
    n = coords.shape[0]

    freqs = (2.0 ** np.arange(L_FREQ, dtype=np.float32)) * np.pi
    pmat = np.concatenate([freqs, freqs])[None, :]
    cmask = np.zeros((1, 36), np.float32)
    for d in range(3):
        cmask[0, 12 * d + 6:12 * d + 12] = 1.0
    def r2(x):
        return x.reshape(1, -1)

    ins = (coords, jnp.asarray(pmat), jnp.asarray(cmask),
           expert_W0, expert_b0, expert_W1, expert_b1, expert_W2, expert_b2,
           expert_W3, expert_b3,
           gate_W0, r2(gate_b0), gate_W1, r2(gate_b1), gate_W2, r2(gate_b2),
           gate_W3, r2(gate_b3), view_embedding, view_embed_proj,
           dec_rgb_W0, r2(dec_rgb_b0), dec_rgb_W1, r2(dec_rgb_b1),
           dec_rgb_W2, r2(dec_rgb_b2), dec_rgb_W3, r2(dec_rgb_b3),
           dec_sigma_W0, r2(dec_sigma_b0), dec_sigma_W1, r2(dec_sigma_b1),
           dec_sigma_W2, r2(dec_sigma_b2), dec_sigma_W3, r2(dec_sigma_b3))

    grid = (n // TILE,)

    def const_spec(x):
        nd = x.ndim
        return pl.BlockSpec(x.shape, lambda i, _nd=nd: (0,) * _nd)

    in_specs = [pl.BlockSpec((TILE, 3), lambda i: (i, 0))] + [const_spec(x) for x in ins[1:]]
    out_specs = [pl.BlockSpec((TILE, 3), lambda i: (i, 0)),
                 pl.BlockSpec((TILE, 1), lambda i: (i, 0))]
    out_shape = [jax.ShapeDtypeStruct((n, 3), jnp.float32),
                 jax.ShapeDtypeStruct((n, 1), jnp.float32)]

    rgb, sigma = pl.pallas_call(
        _body,
        grid=grid,
        in_specs=in_specs,
        out_specs=out_specs,
        out_shape=out_shape,
    )(*ins)
    return (rgb, sigma)
